# baseline (speedup 1.0000x reference)
import jax
import jax.numpy as jnp
from jax import lax
from jax.experimental import pallas as pl
from jax.experimental.pallas import tpu as pltpu
from jax.experimental.pallas import tpu_sc as plsc

_NCHAN = 69
_ROW = 361 * 720
_NW = 32


def _src_channel(c):
    return jnp.where(
        c < 39, c,
        jnp.where(
            c == 39, 65,
            jnp.where(
                c == 40, 66,
                jnp.where(
                    c <= 53, c - 2,
                    jnp.where(c == 54, 67, jnp.where(c <= 67, c - 3, 68))))))


def _body(x_ref, out_ref):
    wid = lax.axis_index("s") * 2 + lax.axis_index("c")
    for k in range(3):
        c = wid + _NW * k

        @pl.when(c < _NCHAN)
        def _copy(c=c):
            s = _src_channel(c)
            pltpu.sync_copy(x_ref.at[s], out_ref.at[c])


def kernel(x):
    x2 = x.reshape(_NCHAN, _ROW)
    out = pl.kernel(
        _body,
        out_type=jax.ShapeDtypeStruct((_NCHAN, _ROW), jnp.float32),
        mesh=plsc.VectorSubcoreMesh(core_axis_name="c", subcore_axis_name="s"),
    )(x2)
    return out.reshape(_NCHAN, 361, 720)


# speedup vs baseline: 2.1057x; 2.1057x over previous
import jax
import jax.numpy as jnp
from jax import lax
from jax.experimental import pallas as pl
from jax.experimental.pallas import tpu as pltpu
from jax.experimental.pallas import tpu_sc as plsc

_NCHAN = 69
_H, _W = 361, 720
_RCHUNK = 56
_NSPLIT = 6
_TAIL = _H - _NSPLIT * _RCHUNK
_NFULL = _NCHAN * _NSPLIT
_NW = 32
_FSTEPS = -(-_NFULL // _NW)
_TSTEPS = -(-_NCHAN // _NW)


def _src_channel(c):
    return jnp.where(
        c < 39, c,
        jnp.where(
            c == 39, 65,
            jnp.where(
                c == 40, 66,
                jnp.where(
                    c <= 53, c - 2,
                    jnp.where(c == 54, 67, jnp.where(c <= 67, c - 3, 68))))))


def _body(x_ref, out_ref, buf0, buf1, tb0, tb1, gsem, ssem, tgsem, tssem):
    wid = lax.axis_index("s") * 2 + lax.axis_index("c")
    bufs = (buf0, buf1)
    tbufs = (tb0, tb1)

    def fitem(i):
        t = wid + _NW * i
        return t, lax.div(t, _NSPLIT), lax.rem(t, _NSPLIT)

    def start_g(i):
        t, c, j = fitem(i)

        @pl.when(t < _NFULL)
        def _():
            pltpu.async_copy(
                x_ref.at[_src_channel(c), pl.ds(j * _RCHUNK, _RCHUNK)],
                bufs[i % 2], gsem.at[i % 2])

    def wait_g(i):
        t, _, _ = fitem(i)

        @pl.when(t < _NFULL)
        def _():
            pltpu.make_async_copy(
                x_ref.at[0, pl.ds(0, _RCHUNK)], bufs[i % 2],
                gsem.at[i % 2]).wait()

    def start_s(i):
        t, c, j = fitem(i)

        @pl.when(t < _NFULL)
        def _():
            pltpu.async_copy(
                bufs[i % 2], out_ref.at[c, pl.ds(j * _RCHUNK, _RCHUNK)],
                ssem.at[i % 2])

    def wait_s(i):
        t, _, _ = fitem(i)

        @pl.when(t < _NFULL)
        def _():
            pltpu.make_async_copy(
                bufs[i % 2], out_ref.at[0, pl.ds(0, _RCHUNK)],
                ssem.at[i % 2]).wait()

    def start_tg(i):
        c = wid + _NW * i

        @pl.when(c < _NCHAN)
        def _():
            pltpu.async_copy(
                x_ref.at[_src_channel(c), pl.ds(_NSPLIT * _RCHUNK, _TAIL)],
                tbufs[i % 2], tgsem.at[i % 2])

    def wait_tg(i):
        c = wid + _NW * i

        @pl.when(c < _NCHAN)
        def _():
            pltpu.make_async_copy(
                x_ref.at[0, pl.ds(0, _TAIL)], tbufs[i % 2],
                tgsem.at[i % 2]).wait()

    def start_ts(i):
        c = wid + _NW * i

        @pl.when(c < _NCHAN)
        def _():
            pltpu.async_copy(
                tbufs[i % 2], out_ref.at[c, pl.ds(_NSPLIT * _RCHUNK, _TAIL)],
                tssem.at[i % 2])

    def wait_ts(i):
        c = wid + _NW * i

        @pl.when(c < _NCHAN)
        def _():
            pltpu.make_async_copy(
                tbufs[i % 2], out_ref.at[0, pl.ds(0, _TAIL)],
                tssem.at[i % 2]).wait()

    start_g(0)
    for i in range(_FSTEPS):
        wait_g(i)
        start_s(i)
        if i + 1 < _FSTEPS:
            if i >= 1:
                wait_s(i - 1)
            start_g(i + 1)

    start_tg(0)
    for i in range(_TSTEPS):
        wait_tg(i)
        start_ts(i)
        if i + 1 < _TSTEPS:
            if i >= 1:
                wait_ts(i - 1)
            start_tg(i + 1)

    wait_s(_FSTEPS - 2)
    wait_s(_FSTEPS - 1)
    wait_ts(_TSTEPS - 2)
    wait_ts(_TSTEPS - 1)


def kernel(x):
    return pl.kernel(
        _body,
        out_type=jax.ShapeDtypeStruct((_NCHAN, _H, _W), jnp.float32),
        mesh=plsc.VectorSubcoreMesh(core_axis_name="c", subcore_axis_name="s"),
        compiler_params=pltpu.CompilerParams(use_tc_tiling_on_sc=False),
        scratch_types=[
            pltpu.VMEM((_RCHUNK, _W), jnp.float32),
            pltpu.VMEM((_RCHUNK, _W), jnp.float32),
            pltpu.VMEM((_TAIL, _W), jnp.float32),
            pltpu.VMEM((_TAIL, _W), jnp.float32),
            pltpu.SemaphoreType.DMA((2,)),
            pltpu.SemaphoreType.DMA((2,)),
            pltpu.SemaphoreType.DMA((2,)),
            pltpu.SemaphoreType.DMA((2,)),
        ],
    )(x)


# speedup vs baseline: 16.7676x; 7.9629x over previous
import jax
import jax.numpy as jnp
from jax import lax
from jax.experimental import pallas as pl
from jax.experimental.pallas import tpu as pltpu
from jax.experimental.pallas import tpu_sc as plsc

_NCHAN = 69
_H, _W = 361, 720
_NWORK = 3
_STEPS = 12


def _src_channel(c):
    return jnp.where(
        c < 39, c,
        jnp.where(
            c == 39, 65,
            jnp.where(
                c == 40, 66,
                jnp.where(
                    c <= 53, c - 2,
                    jnp.where(c == 54, 67, jnp.where(c <= 67, c - 3, 68))))))


def _body(x_ref, out_ref, spm, gsem, ssem):
    cid = lax.axis_index("c")
    sid = lax.axis_index("s")
    nloc = 35 - cid

    @pl.when(sid < _NWORK)
    def _work():
        def chan(k):
            l = sid + _NWORK * k
            return l, cid + 2 * l

        def start_g(k):
            l, c = chan(k)

            @pl.when(l < nloc)
            def _():
                pltpu.async_copy(
                    x_ref.at[_src_channel(c)], spm.at[2 * sid + k % 2],
                    gsem.at[k % 2])

        def wait_g(k):
            l, _ = chan(k)

            @pl.when(l < nloc)
            def _():
                pltpu.make_async_copy(
                    x_ref.at[0], spm.at[2 * sid + k % 2],
                    gsem.at[k % 2]).wait()

        def start_s(k):
            l, c = chan(k)

            @pl.when(l < nloc)
            def _():
                pltpu.async_copy(
                    spm.at[2 * sid + k % 2], out_ref.at[c], ssem.at[k % 2])

        def wait_s(k):
            l, _ = chan(k)

            @pl.when(l < nloc)
            def _():
                pltpu.make_async_copy(
                    spm.at[2 * sid + k % 2], out_ref.at[0],
                    ssem.at[k % 2]).wait()

        start_g(0)
        for k in range(_STEPS):
            wait_g(k)
            start_s(k)
            if k + 1 < _STEPS:
                if k >= 1:
                    wait_s(k - 1)
                start_g(k + 1)
        wait_s(_STEPS - 2)
        wait_s(_STEPS - 1)


def kernel(x):
    return pl.kernel(
        _body,
        out_type=jax.ShapeDtypeStruct((_NCHAN, _H, _W), jnp.float32),
        mesh=plsc.VectorSubcoreMesh(core_axis_name="c", subcore_axis_name="s"),
        scratch_types=[
            pltpu.VMEM_SHARED((2 * _NWORK, _H, _W), jnp.float32),
            pltpu.SemaphoreType.DMA((2,)),
            pltpu.SemaphoreType.DMA((2,)),
        ],
    )(x)


# speedup vs baseline: 49.9524x; 2.9791x over previous
import jax
import jax.numpy as jnp
from jax import lax
from jax.experimental import pallas as pl
from jax.experimental.pallas import tpu as pltpu
from jax.experimental.pallas import tpu_sc as plsc

_NCHAN = 69
_H, _W = 720, 361
_RCHUNK = 144
_NSPLIT = _H // _RCHUNK
_NITEMS = _NCHAN * _NSPLIT
_NW = 32
_STEPS = -(-_NITEMS // _NW)


def _src_channel(c):
    return jnp.where(
        c < 39, c,
        jnp.where(
            c == 39, 65,
            jnp.where(
                c == 40, 66,
                jnp.where(
                    c <= 53, c - 2,
                    jnp.where(c == 54, 67, jnp.where(c <= 67, c - 3, 68))))))


def _body(x_ref, out_ref, buf0, buf1, gsem, ssem):
    wid = lax.axis_index("s") * 2 + lax.axis_index("c")
    bufs = (buf0, buf1)

    def item(i):
        t = wid + _NW * i
        return t, lax.div(t, _NSPLIT), lax.rem(t, _NSPLIT)

    def start_g(i):
        t, c, j = item(i)

        @pl.when(t < _NITEMS)
        def _():
            pltpu.async_copy(
                x_ref.at[_src_channel(c), pl.ds(j * _RCHUNK, _RCHUNK)],
                bufs[i % 2], gsem.at[i % 2])

    def wait_g(i):
        t, _, _ = item(i)

        @pl.when(t < _NITEMS)
        def _():
            pltpu.make_async_copy(
                x_ref.at[0, pl.ds(0, _RCHUNK)], bufs[i % 2],
                gsem.at[i % 2]).wait()

    def start_s(i):
        t, c, j = item(i)

        @pl.when(t < _NITEMS)
        def _():
            pltpu.async_copy(
                bufs[i % 2], out_ref.at[c, pl.ds(j * _RCHUNK, _RCHUNK)],
                ssem.at[i % 2])

    def wait_s(i):
        t, _, _ = item(i)

        @pl.when(t < _NITEMS)
        def _():
            pltpu.make_async_copy(
                bufs[i % 2], out_ref.at[0, pl.ds(0, _RCHUNK)],
                ssem.at[i % 2]).wait()

    start_g(0)
    for i in range(_STEPS):
        wait_g(i)
        start_s(i)
        if i + 1 < _STEPS:
            if i >= 1:
                wait_s(i - 1)
            start_g(i + 1)
    wait_s(_STEPS - 2)
    wait_s(_STEPS - 1)


def kernel(x):
    xt = jnp.swapaxes(x, 1, 2)
    outt = pl.kernel(
        _body,
        out_type=jax.ShapeDtypeStruct((_NCHAN, _H, _W), jnp.float32),
        mesh=plsc.VectorSubcoreMesh(core_axis_name="c", subcore_axis_name="s"),
        scratch_types=[
            pltpu.VMEM((_RCHUNK, _W), jnp.float32),
            pltpu.VMEM((_RCHUNK, _W), jnp.float32),
            pltpu.SemaphoreType.DMA((2,)),
            pltpu.SemaphoreType.DMA((2,)),
        ],
    )(xt)
    return jnp.swapaxes(outt, 1, 2)
